# baseline (speedup 1.0000x reference)
import functools

import jax
import jax.numpy as jnp
from jax import lax
from jax.experimental import pallas as pl
from jax.experimental.pallas import tpu as pltpu
from jax.experimental.pallas import tpu_sc as plsc

NC = 2
NS = 16
NW = NC * NS
L = 16
C = 80
NB = 4


def _mm_body(x_ref, w_ref, o_ref):
    o_ref[...] = jnp.dot(x_ref[...], w_ref[...],
                         preferred_element_type=jnp.float32)


def _add_body(a_ref, b_ref, o_ref):
    o_ref[...] = a_ref[...] + b_ref[...]


def _pad_body(s_ref, d_ref, w_ref, so_ref, do_ref, wo_ref, n_pad_rows):
    e_rows = s_ref.shape[0]
    so_ref[pl.ds(0, e_rows), :] = s_ref[...]
    do_ref[pl.ds(0, e_rows), :] = d_ref[...]
    wo_ref[pl.ds(0, e_rows), :] = w_ref[...]
    so_ref[pl.ds(e_rows, n_pad_rows), :] = jnp.zeros(
        (n_pad_rows, 128), jnp.int32)
    do_ref[pl.ds(e_rows, n_pad_rows), :] = jnp.zeros(
        (n_pad_rows, 128), jnp.int32)
    wo_ref[pl.ds(e_rows, n_pad_rows), :] = jnp.zeros(
        (n_pad_rows, 128), jnp.float32)


def _make_sc_scatter(N, D, EPAD):
    EPS = EPAD // NW
    NCHUNK = EPS // C
    NITER = NCHUNK // (2 * NB)
    RPT = (N // NS) // 8 * 8
    TAIL = N - RPT * NS
    mesh = plsc.VectorSubcoreMesh(core_axis_name="c", subcore_axis_name="s")

    @functools.partial(
        pl.kernel,
        mesh=mesh,
        out_type=jax.ShapeDtypeStruct((NC, N, D), jnp.float32),
        scratch_types=[
            pltpu.VMEM((2 * NB, C), jnp.int32),
            pltpu.VMEM((2 * NB, C), jnp.int32),
            pltpu.VMEM((2 * NB, C), jnp.float32),
            [pltpu.VMEM((C, D), jnp.float32) for _ in range(NB)],
            pltpu.VMEM_SHARED((N, D), jnp.float32),
            [pltpu.SemaphoreType.DMA for _ in range(2 * NB)],
            [pltpu.SemaphoreType.DMA for _ in range(NB)],
        ],
    )
    def sc_fn(presup_hbm, src_hbm, dst_hbm, w_hbm, zeros_hbm, out_hbm,
              srcb, dstb, wbuf, rowsb, acc, msem, gsem):
        cid = lax.axis_index("c")
        sid = lax.axis_index("s")
        wid = sid * NC + cid
        base = wid * EPS

        r0 = pl.multiple_of(sid * RPT, 8)
        pltpu.sync_copy(zeros_hbm.at[pl.ds(r0, RPT)], acc.at[pl.ds(r0, RPT)])

        @pl.when(sid == 0)
        def _zero_tail():
            t0 = RPT * NS
            pltpu.sync_copy(zeros_hbm.at[pl.ds(t0, TAIL)],
                            acc.at[pl.ds(t0, TAIL)])

        plsc.subcore_barrier()

        def issue_meta(i, b):
            off = pl.multiple_of(base + i * C, 8)
            pltpu.async_copy(src_hbm.at[pl.ds(off, C)], srcb.at[b], msem[b])
            pltpu.async_copy(dst_hbm.at[pl.ds(off, C)], dstb.at[b], msem[b])
            pltpu.async_copy(w_hbm.at[pl.ds(off, C)], wbuf.at[b], msem[b])

        def wait_meta(b):
            pltpu.make_async_copy(src_hbm.at[pl.ds(0, C)], srcb.at[b],
                                  msem[b]).wait()
            pltpu.make_async_copy(dst_hbm.at[pl.ds(0, C)], dstb.at[b],
                                  msem[b]).wait()
            pltpu.make_async_copy(w_hbm.at[pl.ds(0, C)], wbuf.at[b],
                                  msem[b]).wait()

        def issue_gather(b, m):
            pltpu.async_copy(presup_hbm.at[srcb.at[m]], rowsb[b], gsem[b])

        def substep(i, b, m):
            pltpu.make_async_copy(presup_hbm.at[pl.ds(0, C)], rowsb[b],
                                  gsem[b]).wait()

            @pl.when(i + (NB - 1) < NCHUNK)
            def _g():
                m2 = (m + NB - 1) % (2 * NB)
                wait_meta(m2)
                issue_gather((b + NB - 1) % NB, m2)

            def grp(g, carry):
                wg = wbuf[m, pl.ds(pl.multiple_of(g * L, 8), L)]
                for k in range(L):
                    e = g * L + k
                    wk = jnp.full((L,), wg[k])
                    for jj in range(D // L):
                        sl = pl.ds(jj * L, L)
                        rowsb[b][e, sl] = rowsb[b][e, sl] * wk
                return carry

            lax.fori_loop(0, C // L, grp, 0)

            pltpu.sync_copy(rowsb[b], acc.at[dstb.at[m]], add=True)

            @pl.when(i + 2 * NB < NCHUNK)
            def _m():
                issue_meta(i + 2 * NB, m)

        for m in range(2 * NB):
            issue_meta(m, m)
        for b in range(NB - 1):
            wait_meta(b)
            issue_gather(b, b)

        def body(j, carry):
            for t in range(2 * NB):
                i = j * (2 * NB) + t
                substep(i, t % NB, t)
            return carry

        lax.fori_loop(0, NITER, body, 0)
        plsc.subcore_barrier()

        pltpu.sync_copy(acc.at[pl.ds(r0, RPT)],
                        out_hbm.at[cid, pl.ds(r0, RPT)])

        @pl.when(sid == 0)
        def _write_tail():
            t0 = RPT * NS
            pltpu.sync_copy(acc.at[pl.ds(t0, TAIL)],
                            out_hbm.at[cid, pl.ds(t0, TAIL)])

    return sc_fn


def kernel(x, edge_index, edge_weight, W0):
    N, D_IN = x.shape
    D_OUT = W0.shape[1]
    E = edge_weight.shape[0]

    BM = 2000
    pre_sup = pl.pallas_call(
        _mm_body,
        grid=(N // BM,),
        in_specs=[
            pl.BlockSpec((BM, D_IN), lambda i: (i, 0)),
            pl.BlockSpec((D_IN, D_OUT), lambda i: (0, 0)),
        ],
        out_specs=pl.BlockSpec((BM, D_OUT), lambda i: (i, 0)),
        out_shape=jax.ShapeDtypeStruct((N, D_OUT), jnp.float32),
    )(x, W0)

    quantum = NW * C * 2 * NB
    e_pad = -(-E // quantum) * quantum
    pad_rows = (e_pad - E) // 128
    e_rows = E // 128
    src2, dst2, w2 = pl.pallas_call(
        functools.partial(_pad_body, n_pad_rows=pad_rows),
        out_shape=[
            jax.ShapeDtypeStruct((e_pad // 128, 128), jnp.int32),
            jax.ShapeDtypeStruct((e_pad // 128, 128), jnp.int32),
            jax.ShapeDtypeStruct((e_pad // 128, 128), jnp.float32),
        ],
    )(edge_index[0].reshape(e_rows, 128), edge_index[1].reshape(e_rows, 128),
      edge_weight.reshape(e_rows, 128))
    src = src2.reshape(e_pad)
    dst = dst2.reshape(e_pad)
    w = w2.reshape(e_pad)
    zeros = jnp.zeros((N, D_OUT), jnp.float32)

    sc_fn = _make_sc_scatter(N, D_OUT, e_pad)
    partials = sc_fn(pre_sup, src, dst, w, zeros)

    out = pl.pallas_call(
        _add_body,
        grid=(N // BM,),
        in_specs=[
            pl.BlockSpec((BM, D_OUT), lambda i: (i, 0)),
            pl.BlockSpec((BM, D_OUT), lambda i: (i, 0)),
        ],
        out_specs=pl.BlockSpec((BM, D_OUT), lambda i: (i, 0)),
        out_shape=jax.ShapeDtypeStruct((N, D_OUT), jnp.float32),
    )(partials[0], partials[1])
    return out


# speedup vs baseline: 1.8492x; 1.8492x over previous
import functools

import jax
import jax.numpy as jnp
from jax import lax
from jax.experimental import pallas as pl
from jax.experimental.pallas import tpu as pltpu
from jax.experimental.pallas import tpu_sc as plsc

NC = 2
NS = 16
NW = NC * NS
L = 16
C = 112
NB = 3


def _mm_body(x_ref, w_ref, o_ref):
    o_ref[...] = jnp.dot(x_ref[...], w_ref[...],
                         preferred_element_type=jnp.float32)


def _add_body(a_ref, b_ref, o_ref):
    o_ref[...] = a_ref[...] + b_ref[...]


def _pad_body(s_ref, d_ref, w_ref, so_ref, do_ref, wo_ref, n_pad_rows):
    e_rows = s_ref.shape[0]
    so_ref[pl.ds(0, e_rows), :] = s_ref[...]
    do_ref[pl.ds(0, e_rows), :] = d_ref[...]
    wo_ref[pl.ds(0, e_rows), :] = w_ref[...]
    so_ref[pl.ds(e_rows, n_pad_rows), :] = jnp.zeros(
        (n_pad_rows, 128), jnp.int32)
    do_ref[pl.ds(e_rows, n_pad_rows), :] = jnp.zeros(
        (n_pad_rows, 128), jnp.int32)
    wo_ref[pl.ds(e_rows, n_pad_rows), :] = jnp.zeros(
        (n_pad_rows, 128), jnp.float32)


def _make_sc_scatter(N, D, EPAD):
    EPS = EPAD // NW
    NCHUNK = EPS // C
    NITER = NCHUNK // (2 * NB)
    RPT = (N // NS) // 8 * 8
    TAIL = N - RPT * NS
    mesh = plsc.VectorSubcoreMesh(core_axis_name="c", subcore_axis_name="s")

    @functools.partial(
        pl.kernel,
        mesh=mesh,
        out_type=jax.ShapeDtypeStruct((NC, N, D), jnp.float32),
        scratch_types=[
            pltpu.VMEM((2 * NB, C), jnp.int32),
            pltpu.VMEM((2 * NB, C), jnp.int32),
            pltpu.VMEM((2 * NB, C), jnp.float32),
            [pltpu.VMEM((C, D), jnp.float32) for _ in range(NB)],
            pltpu.VMEM_SHARED((N, D), jnp.float32),
            [pltpu.SemaphoreType.DMA for _ in range(2 * NB)],
            [pltpu.SemaphoreType.DMA for _ in range(NB)],
            [pltpu.SemaphoreType.DMA for _ in range(NB)],
        ],
    )
    def sc_fn(presup_hbm, src_hbm, dst_hbm, w_hbm, zeros_hbm, out_hbm,
              srcb, dstb, wbuf, rowsb, acc, msem, gsem, ssem):
        cid = lax.axis_index("c")
        sid = lax.axis_index("s")
        wid = sid * NC + cid
        base = wid * EPS

        r0 = pl.multiple_of(sid * RPT, 8)
        pltpu.sync_copy(zeros_hbm.at[pl.ds(r0, RPT)], acc.at[pl.ds(r0, RPT)])

        @pl.when(sid == 0)
        def _zero_tail():
            t0 = RPT * NS
            pltpu.sync_copy(zeros_hbm.at[pl.ds(t0, TAIL)],
                            acc.at[pl.ds(t0, TAIL)])

        plsc.subcore_barrier()

        def issue_meta(i, b):
            off = pl.multiple_of(base + i * C, 8)
            pltpu.async_copy(src_hbm.at[pl.ds(off, C)], srcb.at[b], msem[b])
            pltpu.async_copy(dst_hbm.at[pl.ds(off, C)], dstb.at[b], msem[b])
            pltpu.async_copy(w_hbm.at[pl.ds(off, C)], wbuf.at[b], msem[b])

        def wait_meta(b):
            pltpu.make_async_copy(src_hbm.at[pl.ds(0, C)], srcb.at[b],
                                  msem[b]).wait()
            pltpu.make_async_copy(dst_hbm.at[pl.ds(0, C)], dstb.at[b],
                                  msem[b]).wait()
            pltpu.make_async_copy(w_hbm.at[pl.ds(0, C)], wbuf.at[b],
                                  msem[b]).wait()

        def issue_gather(b, m):
            pltpu.async_copy(presup_hbm.at[srcb.at[m]], rowsb[b], gsem[b])

        def substep(i, b, m):
            pltpu.make_async_copy(presup_hbm.at[pl.ds(0, C)], rowsb[b],
                                  gsem[b]).wait()

            @pl.when(i + 2 < NCHUNK)
            def _g():
                b2 = (b + 2) % NB
                m2 = (m + 2) % (2 * NB)

                @pl.when(i >= 1)
                def _w():
                    pltpu.make_async_copy(rowsb[b2], acc.at[pl.ds(0, C)],
                                          ssem[b2]).wait()

                    @pl.when(i + 5 < NCHUNK)
                    def _m():
                        issue_meta(i + 5, (m + 5) % (2 * NB))

                wait_meta(m2)
                issue_gather(b2, m2)

            def grp(g, carry):
                wg = wbuf[m, pl.ds(pl.multiple_of(g * L, 8), L)]
                for k in range(L):
                    e = g * L + k
                    wk = jnp.full((L,), wg[k])
                    for jj in range(D // L):
                        sl = pl.ds(jj * L, L)
                        rowsb[b][e, sl] = rowsb[b][e, sl] * wk
                return carry

            lax.fori_loop(0, C // L, grp, 0)

            pltpu.async_copy(rowsb[b], acc.at[dstb.at[m]], ssem[b], add=True)

        for m in range(2 * NB):
            issue_meta(m, m)
        wait_meta(0)
        issue_gather(0, 0)
        wait_meta(1)
        issue_gather(1, 1)

        def body(j, carry):
            for t in range(2 * NB):
                i = j * (2 * NB) + t
                substep(i, t % NB, t)
            return carry

        lax.fori_loop(0, NITER, body, 0)
        for b in range(NB):
            pltpu.make_async_copy(rowsb[b], acc.at[pl.ds(0, C)],
                                  ssem[b]).wait()
        plsc.subcore_barrier()

        pltpu.sync_copy(acc.at[pl.ds(r0, RPT)],
                        out_hbm.at[cid, pl.ds(r0, RPT)])

        @pl.when(sid == 0)
        def _write_tail():
            t0 = RPT * NS
            pltpu.sync_copy(acc.at[pl.ds(t0, TAIL)],
                            out_hbm.at[cid, pl.ds(t0, TAIL)])

    return sc_fn


def kernel(x, edge_index, edge_weight, W0):
    N, D_IN = x.shape
    D_OUT = W0.shape[1]
    E = edge_weight.shape[0]

    BM = 2000
    pre_sup = pl.pallas_call(
        _mm_body,
        grid=(N // BM,),
        in_specs=[
            pl.BlockSpec((BM, D_IN), lambda i: (i, 0)),
            pl.BlockSpec((D_IN, D_OUT), lambda i: (0, 0)),
        ],
        out_specs=pl.BlockSpec((BM, D_OUT), lambda i: (i, 0)),
        out_shape=jax.ShapeDtypeStruct((N, D_OUT), jnp.float32),
    )(x, W0)

    quantum = NW * C * 2 * NB
    e_pad = -(-E // quantum) * quantum
    pad_rows = (e_pad - E) // 128
    e_rows = E // 128
    src2, dst2, w2 = pl.pallas_call(
        functools.partial(_pad_body, n_pad_rows=pad_rows),
        out_shape=[
            jax.ShapeDtypeStruct((e_pad // 128, 128), jnp.int32),
            jax.ShapeDtypeStruct((e_pad // 128, 128), jnp.int32),
            jax.ShapeDtypeStruct((e_pad // 128, 128), jnp.float32),
        ],
    )(edge_index[0].reshape(e_rows, 128), edge_index[1].reshape(e_rows, 128),
      edge_weight.reshape(e_rows, 128))
    src = src2.reshape(e_pad)
    dst = dst2.reshape(e_pad)
    w = w2.reshape(e_pad)
    zeros = jnp.zeros((N, D_OUT), jnp.float32)

    sc_fn = _make_sc_scatter(N, D_OUT, e_pad)
    partials = sc_fn(pre_sup, src, dst, w, zeros)

    out = pl.pallas_call(
        _add_body,
        grid=(N // BM,),
        in_specs=[
            pl.BlockSpec((BM, D_OUT), lambda i: (i, 0)),
            pl.BlockSpec((BM, D_OUT), lambda i: (i, 0)),
        ],
        out_specs=pl.BlockSpec((BM, D_OUT), lambda i: (i, 0)),
        out_shape=jax.ShapeDtypeStruct((N, D_OUT), jnp.float32),
    )(partials[0], partials[1])
    return out


# speedup vs baseline: 1.8766x; 1.0148x over previous
import functools

import jax
import jax.numpy as jnp
from jax import lax
from jax.experimental import pallas as pl
from jax.experimental.pallas import tpu as pltpu
from jax.experimental.pallas import tpu_sc as plsc

NC = 2
NS = 16
NW = NC * NS
L = 16
C = 112
NB = 3


def _mm_body(x_ref, w_ref, o_ref):
    o_ref[...] = jnp.dot(x_ref[...], w_ref[...],
                         preferred_element_type=jnp.float32)


def _add_body(a_ref, b_ref, o_ref):
    o_ref[...] = a_ref[...] + b_ref[...]


def _pad_body(s_ref, d_ref, w_ref, so_ref, do_ref, wo_ref, n_pad_rows):
    e_rows = s_ref.shape[0]
    so_ref[pl.ds(0, e_rows), :] = s_ref[...]
    do_ref[pl.ds(0, e_rows), :] = d_ref[...]
    wo_ref[pl.ds(0, e_rows), :] = w_ref[...]
    so_ref[pl.ds(e_rows, n_pad_rows), :] = jnp.zeros(
        (n_pad_rows, 128), jnp.int32)
    do_ref[pl.ds(e_rows, n_pad_rows), :] = jnp.zeros(
        (n_pad_rows, 128), jnp.int32)
    wo_ref[pl.ds(e_rows, n_pad_rows), :] = jnp.zeros(
        (n_pad_rows, 128), jnp.float32)


def _make_sc_scatter(N, D, EPAD):
    EPS = EPAD // NW
    NCHUNK = EPS // C
    NITER = NCHUNK // (2 * NB)
    RPT = (N // NS) // 8 * 8
    TAIL = N - RPT * NS
    mesh = plsc.VectorSubcoreMesh(core_axis_name="c", subcore_axis_name="s")

    @functools.partial(
        pl.kernel,
        mesh=mesh,
        out_type=jax.ShapeDtypeStruct((NC, N, D), jnp.float32),
        scratch_types=[
            pltpu.VMEM((2 * NB, C), jnp.int32),
            pltpu.VMEM((2 * NB, C), jnp.int32),
            pltpu.VMEM((2 * NB, C), jnp.float32),
            [pltpu.VMEM((C, D), jnp.float32) for _ in range(NB)],
            pltpu.VMEM_SHARED((N, D), jnp.float32),
            [pltpu.SemaphoreType.DMA for _ in range(2 * NB)],
            [pltpu.SemaphoreType.DMA for _ in range(NB)],
            [pltpu.SemaphoreType.DMA for _ in range(NB)],
        ],
    )
    def sc_fn(presup_hbm, src_hbm, dst_hbm, w_hbm, zeros_hbm, out_hbm,
              srcb, dstb, wbuf, rowsb, acc, msem, gsem, ssem):
        cid = lax.axis_index("c")
        sid = lax.axis_index("s")
        wid = sid * NC + cid
        base = wid * EPS

        r0 = pl.multiple_of(sid * RPT, 8)
        pltpu.sync_copy(zeros_hbm.at[pl.ds(r0, RPT)], acc.at[pl.ds(r0, RPT)])

        @pl.when(sid == 0)
        def _zero_tail():
            t0 = RPT * NS
            pltpu.sync_copy(zeros_hbm.at[pl.ds(t0, TAIL)],
                            acc.at[pl.ds(t0, TAIL)])

        plsc.subcore_barrier()

        def issue_meta(i, b):
            off = pl.multiple_of(base + i * C, 8)
            pltpu.async_copy(src_hbm.at[pl.ds(off, C)], srcb.at[b], msem[b])
            pltpu.async_copy(dst_hbm.at[pl.ds(off, C)], dstb.at[b], msem[b])
            pltpu.async_copy(w_hbm.at[pl.ds(off, C)], wbuf.at[b], msem[b])

        def wait_meta(b):
            pltpu.make_async_copy(src_hbm.at[pl.ds(0, C)], srcb.at[b],
                                  msem[b]).wait()
            pltpu.make_async_copy(dst_hbm.at[pl.ds(0, C)], dstb.at[b],
                                  msem[b]).wait()
            pltpu.make_async_copy(w_hbm.at[pl.ds(0, C)], wbuf.at[b],
                                  msem[b]).wait()

        def issue_gather(b, m):
            h = C // 2
            pltpu.async_copy(presup_hbm.at[srcb.at[m, pl.ds(0, h)]],
                             rowsb[b].at[pl.ds(0, h)], gsem[b])
            pltpu.async_copy(presup_hbm.at[srcb.at[m, pl.ds(h, h)]],
                             rowsb[b].at[pl.ds(h, h)], gsem[b])

        def substep(i, b, m):
            pltpu.make_async_copy(presup_hbm.at[pl.ds(0, C)], rowsb[b],
                                  gsem[b]).wait()

            @pl.when(i + 2 < NCHUNK)
            def _g():
                b2 = (b + 2) % NB
                m2 = (m + 2) % (2 * NB)

                @pl.when(i >= 1)
                def _w():
                    pltpu.make_async_copy(rowsb[b2], acc.at[pl.ds(0, C)],
                                          ssem[b2]).wait()

                    @pl.when(i + 5 < NCHUNK)
                    def _m():
                        issue_meta(i + 5, (m + 5) % (2 * NB))

                wait_meta(m2)
                issue_gather(b2, m2)

            def grp(g, carry):
                wg = wbuf[m, pl.ds(pl.multiple_of(g * L, 8), L)]
                for k in range(L):
                    e = g * L + k
                    wk = jnp.full((L,), wg[k])
                    for jj in range(D // L):
                        sl = pl.ds(jj * L, L)
                        rowsb[b][e, sl] = rowsb[b][e, sl] * wk
                return carry

            lax.fori_loop(0, C // L, grp, 0)

            pltpu.async_copy(rowsb[b], acc.at[dstb.at[m]], ssem[b], add=True)

        for m in range(2 * NB):
            issue_meta(m, m)
        wait_meta(0)
        issue_gather(0, 0)
        wait_meta(1)
        issue_gather(1, 1)

        def body(j, carry):
            for t in range(2 * NB):
                i = j * (2 * NB) + t
                substep(i, t % NB, t)
            return carry

        lax.fori_loop(0, NITER, body, 0)
        for b in range(NB):
            pltpu.make_async_copy(rowsb[b], acc.at[pl.ds(0, C)],
                                  ssem[b]).wait()
        plsc.subcore_barrier()

        pltpu.sync_copy(acc.at[pl.ds(r0, RPT)],
                        out_hbm.at[cid, pl.ds(r0, RPT)])

        @pl.when(sid == 0)
        def _write_tail():
            t0 = RPT * NS
            pltpu.sync_copy(acc.at[pl.ds(t0, TAIL)],
                            out_hbm.at[cid, pl.ds(t0, TAIL)])

    return sc_fn


def kernel(x, edge_index, edge_weight, W0):
    N, D_IN = x.shape
    D_OUT = W0.shape[1]
    E = edge_weight.shape[0]

    BM = 2000
    pre_sup = pl.pallas_call(
        _mm_body,
        grid=(N // BM,),
        in_specs=[
            pl.BlockSpec((BM, D_IN), lambda i: (i, 0)),
            pl.BlockSpec((D_IN, D_OUT), lambda i: (0, 0)),
        ],
        out_specs=pl.BlockSpec((BM, D_OUT), lambda i: (i, 0)),
        out_shape=jax.ShapeDtypeStruct((N, D_OUT), jnp.float32),
    )(x, W0)

    quantum = NW * C * 2 * NB
    e_pad = -(-E // quantum) * quantum
    pad_rows = (e_pad - E) // 128
    e_rows = E // 128
    src2, dst2, w2 = pl.pallas_call(
        functools.partial(_pad_body, n_pad_rows=pad_rows),
        out_shape=[
            jax.ShapeDtypeStruct((e_pad // 128, 128), jnp.int32),
            jax.ShapeDtypeStruct((e_pad // 128, 128), jnp.int32),
            jax.ShapeDtypeStruct((e_pad // 128, 128), jnp.float32),
        ],
    )(edge_index[0].reshape(e_rows, 128), edge_index[1].reshape(e_rows, 128),
      edge_weight.reshape(e_rows, 128))
    src = src2.reshape(e_pad)
    dst = dst2.reshape(e_pad)
    w = w2.reshape(e_pad)
    zeros = jnp.zeros((N, D_OUT), jnp.float32)

    sc_fn = _make_sc_scatter(N, D_OUT, e_pad)
    partials = sc_fn(pre_sup, src, dst, w, zeros)

    out = pl.pallas_call(
        _add_body,
        grid=(N // BM,),
        in_specs=[
            pl.BlockSpec((BM, D_OUT), lambda i: (i, 0)),
            pl.BlockSpec((BM, D_OUT), lambda i: (i, 0)),
        ],
        out_specs=pl.BlockSpec((BM, D_OUT), lambda i: (i, 0)),
        out_shape=jax.ShapeDtypeStruct((N, D_OUT), jnp.float32),
    )(partials[0], partials[1])
    return out
